# baseline (speedup 1.0000x reference)
import functools

import jax
import jax.numpy as jnp
from jax import lax
from jax.experimental import pallas as pl
from jax.experimental.pallas import tpu as pltpu
from jax.experimental.pallas import tpu_sc as plsc

_NC = 2
_NS = 16
_NW = _NC * _NS
_L = 16

_N = 10000 * 128
_PER_W = _N // _NW
_ACCS = 10
_ITERS = _PER_W // (_L * _ACCS)


def _sc_sumsq_body(x_hbm, out_hbm, buf, part):
    cid = lax.axis_index("c")
    sid = lax.axis_index("s")
    wid = sid * _NC + cid
    base = wid * _PER_W
    pltpu.sync_copy(x_hbm.at[pl.ds(base, _PER_W)], buf)

    def body(i, accs):
        off = i * (_L * _ACCS)
        new = []
        for j in range(_ACCS):
            v = buf[pl.ds(off + j * _L, _L)]
            new.append(accs[j] + v * v)
        return tuple(new)

    zero = jnp.zeros((_L,), jnp.float32)
    accs = lax.fori_loop(0, _ITERS, body, (zero,) * _ACCS)
    total = accs[0]
    for j in range(1, _ACCS):
        total = total + accs[j]
    part[...] = total
    pltpu.sync_copy(part, out_hbm.at[wid])


_sc_sumsq = functools.partial(
    pl.kernel,
    out_type=jax.ShapeDtypeStruct((_NW, _L), jnp.float32),
    mesh=plsc.VectorSubcoreMesh(core_axis_name="c", subcore_axis_name="s"),
    scratch_types=[
        pltpu.VMEM((_PER_W,), jnp.float32),
        pltpu.VMEM((_L,), jnp.float32),
    ],
)(_sc_sumsq_body)


def _tc_finish_body(p_ref, g_ref, o_ref):
    s = jnp.sum(p_ref[...])
    o_ref[0] = jnp.sqrt(s)
    o_ref[1] = g_ref[1]
    o_ref[2] = g_ref[2]


def _tc_finish(partials, g):
    return pl.pallas_call(
        _tc_finish_body,
        out_shape=jax.ShapeDtypeStruct((3,), jnp.float32),
        in_specs=[
            pl.BlockSpec(memory_space=pltpu.VMEM),
            pl.BlockSpec(memory_space=pltpu.SMEM),
        ],
        out_specs=pl.BlockSpec(memory_space=pltpu.SMEM),
    )(partials, g)


def kernel(vertex_attr, edgeij_pair, edge_attr, g, batch):
    x = vertex_attr.reshape(-1)
    partials = _sc_sumsq(x)
    return _tc_finish(partials, g)
